# Initial kernel scaffold; baseline (speedup 1.0000x reference)
import jax
import jax.numpy as jnp
from jax.experimental import pallas as pl


def kernel(edge_index, node_features, W1, b1, W2, b2):
    raise NotImplementedError("write your pallas kernel here")


# speedup vs baseline: 12.8032x; 12.8032x over previous
import functools

import jax
import jax.numpy as jnp
from jax import lax
from jax.experimental import pallas as pl
from jax.experimental.pallas import tpu as pltpu
from jax.experimental.pallas import tpu_sc as plsc

N = 10000
HID = 128
E = 320000
NC = 2
NS = 16
NW = NC * NS
LANES = 16

CHUNK = 80
CHUNKS_PER_TILE = E // (NW * CHUNK)
ROWB = 80
TILE_ROWS = 640

_mesh = plsc.VectorSubcoreMesh(
    core_axis_name="c", subcore_axis_name="s", num_cores=NC, num_subcores=NS
)


def _row_chunks(s):
    return s * TILE_ROWS, jnp.where(s == NS - 1, 5, TILE_ROWS // ROWB)


def _degree_body(dst_hbm, cnt_hbm, dst_v, ones_v, zbuf, acc):
    c = lax.axis_index("c")
    s = lax.axis_index("s")
    w = c * NS + s
    row0, nt = _row_chunks(s)

    def _fill(i, _):
        for j in range(HID // LANES):
            zbuf[i, pl.ds(j * LANES, LANES)] = jnp.zeros((LANES,), jnp.float32)
            ones_v[i, pl.ds(j * LANES, LANES)] = jnp.ones((LANES,), jnp.float32)
        return 0

    lax.fori_loop(0, ROWB, _fill, 0)

    def _zero(t, _):
        pltpu.sync_copy(zbuf, acc.at[pl.ds(row0 + t * ROWB, ROWB)])
        return 0

    lax.fori_loop(0, nt, _zero, 0)
    plsc.subcore_barrier()

    def _body(t, _):
        base = w * (CHUNKS_PER_TILE * CHUNK) + t * CHUNK
        pltpu.sync_copy(dst_hbm.at[pl.ds(base, CHUNK)], dst_v)
        pltpu.sync_copy(ones_v, acc.at[dst_v], add=True)
        return 0

    lax.fori_loop(0, CHUNKS_PER_TILE, _body, 0)
    plsc.subcore_barrier()

    def _dump(t, _):
        r0 = row0 + t * ROWB
        pltpu.sync_copy(acc.at[pl.ds(r0, ROWB)], ones_v)
        pltpu.sync_copy(ones_v, cnt_hbm.at[pl.ds(c * N + r0, ROWB)])
        return 0

    lax.fori_loop(0, nt, _dump, 0)


def _build_degree(interpret=False):
    return pl.kernel(
        _degree_body,
        out_type=jax.ShapeDtypeStruct((NC * N, HID), jnp.float32),
        mesh=_mesh,
        scratch_types=[
            pltpu.VMEM((CHUNK,), jnp.int32),
            pltpu.VMEM((CHUNK, HID), jnp.float32),
            pltpu.VMEM((ROWB, HID), jnp.float32),
            pltpu.VMEM_SHARED((N, HID), jnp.float32),
        ],
        interpret=interpret,
    )


def _edges_body(src_hbm, dst_hbm, hp_hbm, out_hbm, src_v, dst_v, rows_v, zbuf, acc, sem):
    c = lax.axis_index("c")
    s = lax.axis_index("s")
    w = c * NS + s
    row0, nt = _row_chunks(s)

    def _fill(i, _):
        for j in range(HID // LANES):
            zbuf[i, pl.ds(j * LANES, LANES)] = jnp.zeros((LANES,), jnp.float32)
        return 0

    lax.fori_loop(0, ROWB, _fill, 0)

    def _zero(t, _):
        pltpu.sync_copy(zbuf, acc.at[pl.ds(row0 + t * ROWB, ROWB)])
        return 0

    lax.fori_loop(0, nt, _zero, 0)
    plsc.subcore_barrier()

    def _body(t, _):
        base = w * (CHUNKS_PER_TILE * CHUNK) + t * CHUNK
        pltpu.sync_copy(src_hbm.at[pl.ds(base, CHUNK)], src_v)
        pltpu.sync_copy(dst_hbm.at[pl.ds(base, CHUNK)], dst_v)
        pltpu.async_copy(hp_hbm.at[src_v], rows_v, sem).wait()
        pltpu.sync_copy(rows_v, acc.at[dst_v], add=True)
        return 0

    lax.fori_loop(0, CHUNKS_PER_TILE, _body, 0)
    plsc.subcore_barrier()

    def _dump(t, _):
        r0 = row0 + t * ROWB
        pltpu.sync_copy(acc.at[pl.ds(r0, ROWB)], rows_v)
        pltpu.sync_copy(rows_v, out_hbm.at[pl.ds(c * N + r0, ROWB)])
        return 0

    lax.fori_loop(0, nt, _dump, 0)


def _build_edges(interpret=False):
    return pl.kernel(
        _edges_body,
        out_type=jax.ShapeDtypeStruct((NC * N, HID), jnp.float32),
        mesh=_mesh,
        scratch_types=[
            pltpu.VMEM((CHUNK,), jnp.int32),
            pltpu.VMEM((CHUNK,), jnp.int32),
            pltpu.VMEM((CHUNK, HID), jnp.float32),
            pltpu.VMEM((ROWB, HID), jnp.float32),
            pltpu.VMEM_SHARED((N, HID), jnp.float32),
            pltpu.SemaphoreType.DMA,
        ],
        interpret=interpret,
    )


_degree_sc = _build_degree()
_edges_sc = _build_edges()


BR = 1000


def _dense1_body(x_ref, w_ref, cnt_ref, h_ref, dis_ref):
    deg = cnt_ref[0][:, 0] + cnt_ref[1][:, 0] + 1.0
    dis = lax.rsqrt(deg)[:, None]
    h = jnp.dot(x_ref[...], w_ref[...], preferred_element_type=jnp.float32)
    h_ref[...] = h * dis
    dis_ref[...] = dis


_dense1 = pl.pallas_call(
    _dense1_body,
    grid=(N // BR,),
    in_specs=[
        pl.BlockSpec((BR, HID), lambda i: (i, 0)),
        pl.BlockSpec((HID, HID), lambda i: (0, 0)),
        pl.BlockSpec((NC, BR, HID), lambda i: (0, i, 0)),
    ],
    out_specs=[
        pl.BlockSpec((BR, HID), lambda i: (i, 0)),
        pl.BlockSpec((BR, 1), lambda i: (i, 0)),
    ],
    out_shape=[
        jax.ShapeDtypeStruct((N, HID), jnp.float32),
        jax.ShapeDtypeStruct((N, 1), jnp.float32),
    ],
)


def _dense2_body(a_ref, hp_ref, dis_ref, b_ref, w_ref, h_ref):
    dis = dis_ref[...]
    seg = a_ref[0] + a_ref[1] + hp_ref[...]
    x2 = jnp.maximum(seg * dis + b_ref[...], 0.0)
    h_ref[...] = jnp.dot(x2, w_ref[...], preferred_element_type=jnp.float32) * dis


_dense2 = pl.pallas_call(
    _dense2_body,
    grid=(N // BR,),
    in_specs=[
        pl.BlockSpec((NC, BR, HID), lambda i: (0, i, 0)),
        pl.BlockSpec((BR, HID), lambda i: (i, 0)),
        pl.BlockSpec((BR, 1), lambda i: (i, 0)),
        pl.BlockSpec((1, HID), lambda i: (0, 0)),
        pl.BlockSpec((HID, HID), lambda i: (0, 0)),
    ],
    out_specs=pl.BlockSpec((BR, HID), lambda i: (i, 0)),
    out_shape=jax.ShapeDtypeStruct((N, HID), jnp.float32),
)


def _dense3_body(a_ref, hp_ref, dis_ref, b_ref, o_ref):
    seg = a_ref[0] + a_ref[1] + hp_ref[...]
    o_ref[...] = jnp.maximum(seg * dis_ref[...] + b_ref[...], 0.0)


_dense3 = pl.pallas_call(
    _dense3_body,
    grid=(N // BR,),
    in_specs=[
        pl.BlockSpec((NC, BR, HID), lambda i: (0, i, 0)),
        pl.BlockSpec((BR, HID), lambda i: (i, 0)),
        pl.BlockSpec((BR, 1), lambda i: (i, 0)),
        pl.BlockSpec((1, HID), lambda i: (0, 0)),
    ],
    out_specs=pl.BlockSpec((BR, HID), lambda i: (i, 0)),
    out_shape=jax.ShapeDtypeStruct((N, HID), jnp.float32),
)


def kernel(edge_index, node_features, W1, b1, W2, b2):
    ei = edge_index.astype(jnp.int32)
    src, dst = ei[0], ei[1]
    cnt = _degree_sc(dst).reshape(NC, N, HID)
    h1p, dis = _dense1(node_features, W1, cnt)
    a1 = _edges_sc(src, dst, h1p).reshape(NC, N, HID)
    h2p = _dense2(a1, h1p, dis, b1.reshape(1, HID), W2)
    a2 = _edges_sc(src, dst, h2p).reshape(NC, N, HID)
    return _dense3(a2, h2p, dis, b2.reshape(1, HID))
